# baseline (device time: 34805 ns/iter reference)
import jax
import jax.numpy as jnp
from jax import lax
from jax.experimental import pallas as pl
from jax.experimental.pallas import tpu as pltpu

Y_SIZE = 2
GRID = 8


def kernel(x):
    m, n = x.shape
    n_global = n * Y_SIZE
    mc = m // GRID

    def body(x_ref, out_ref, acc, send_buf, recv_buf, send_sem, recv_sem):
        g = pl.program_id(0)
        my_x = lax.axis_index("x")
        my_y = lax.axis_index("y")
        nbr = (my_x, 1 - my_y)

        @pl.when(g == 0)
        def _():
            barrier_sem = pltpu.get_barrier_semaphore()
            pl.semaphore_signal(
                barrier_sem,
                inc=1,
                device_id=nbr,
                device_id_type=pl.DeviceIdType.MESH,
            )
            pl.semaphore_wait(barrier_sem, 1)

        s = x_ref[:, 0:128]
        for c in range(128, n, 128):
            s = s + x_ref[:, c : c + 128]
        acc[pl.ds(g * mc, mc), :] = s

        @pl.when(g == GRID - 1)
        def _():
            send_buf[:, :] = jnp.sum(acc[:, :], axis=1, keepdims=True)
            rdma = pltpu.make_async_remote_copy(
                src_ref=send_buf,
                dst_ref=recv_buf,
                send_sem=send_sem,
                recv_sem=recv_sem,
                device_id=nbr,
                device_id_type=pl.DeviceIdType.MESH,
            )
            rdma.start()
            rdma.wait()
            out_ref[:, :] = (send_buf[:, :] + recv_buf[:, :]) * (1.0 / n_global)

    return pl.pallas_call(
        body,
        grid=(GRID,),
        out_shape=jax.ShapeDtypeStruct((m, 1), jnp.float32),
        in_specs=[
            pl.BlockSpec((mc, n), lambda g: (g, 0), memory_space=pltpu.VMEM)
        ],
        out_specs=pl.BlockSpec((m, 1), lambda g: (0, 0), memory_space=pltpu.VMEM),
        scratch_shapes=[
            pltpu.VMEM((m, 128), jnp.float32),
            pltpu.VMEM((m, 1), jnp.float32),
            pltpu.VMEM((m, 1), jnp.float32),
            pltpu.SemaphoreType.DMA,
            pltpu.SemaphoreType.DMA,
        ],
        compiler_params=pltpu.CompilerParams(collective_id=0),
    )(x)


# device time: 11596 ns/iter; 3.0015x vs baseline; 3.0015x over previous
import jax
import jax.numpy as jnp
from jax import lax
from jax.experimental import pallas as pl
from jax.experimental.pallas import tpu as pltpu

Y_SIZE = 2
GRID = 8


def kernel(x):
    m, n = x.shape
    n_global = n * Y_SIZE
    mc = m // GRID
    mr = m // 128

    def body(x_ref, out_ref, acc, send_buf, recv_buf, send_sem, recv_sem):
        g = pl.program_id(0)
        my_x = lax.axis_index("x")
        my_y = lax.axis_index("y")
        nbr = (my_x, 1 - my_y)

        @pl.when(g == 0)
        def _():
            barrier_sem = pltpu.get_barrier_semaphore()
            pl.semaphore_signal(
                barrier_sem,
                inc=1,
                device_id=nbr,
                device_id_type=pl.DeviceIdType.MESH,
            )
            pl.semaphore_wait(barrier_sem, 1)

        s = x_ref[:, 0:128]
        for c in range(128, n, 128):
            s = s + x_ref[:, c : c + 128]
        acc[pl.ds(g * mc, mc), :] = s

        @pl.when(g == GRID - 1)
        def _():
            for i in range(mr):
                send_buf[i, :] = jnp.sum(
                    acc[pl.ds(i * 128, 128), :], axis=1
                )
            rdma = pltpu.make_async_remote_copy(
                src_ref=send_buf,
                dst_ref=recv_buf,
                send_sem=send_sem,
                recv_sem=recv_sem,
                device_id=nbr,
                device_id_type=pl.DeviceIdType.MESH,
            )
            rdma.start()
            rdma.wait()
            out_ref[:, :] = (send_buf[:, :] + recv_buf[:, :]) * (1.0 / n_global)

    compact = pl.pallas_call(
        body,
        grid=(GRID,),
        out_shape=jax.ShapeDtypeStruct((mr, 128), jnp.float32),
        in_specs=[
            pl.BlockSpec((mc, n), lambda g: (g, 0), memory_space=pltpu.VMEM)
        ],
        out_specs=pl.BlockSpec(
            (mr, 128), lambda g: (0, 0), memory_space=pltpu.VMEM
        ),
        scratch_shapes=[
            pltpu.VMEM((m, 128), jnp.float32),
            pltpu.VMEM((mr, 128), jnp.float32),
            pltpu.VMEM((mr, 128), jnp.float32),
            pltpu.SemaphoreType.DMA,
            pltpu.SemaphoreType.DMA,
        ],
        compiler_params=pltpu.CompilerParams(collective_id=0),
    )(x)
    return compact.reshape(m, 1)
